# baseline (speedup 1.0000x reference)
import functools

import jax
import jax.numpy as jnp
from jax import lax
from jax.experimental import pallas as pl
from jax.experimental.pallas import tpu as pltpu
from jax.experimental.pallas import tpu_sc as plsc

D = 64
MAXLEN = 200
NC = 2
NS = 16
NW = NC * NS
G = 4
GB = 100
CHUNK = G * GB


@functools.partial(jax.jit, static_argnums=(3,))
def _run(x_r, token_table, pos_table, total_rows):
    chunks = total_rows // (NW * CHUNK)

    mesh = plsc.VectorSubcoreMesh(core_axis_name="c", subcore_axis_name="s")

    @functools.partial(
        pl.kernel,
        mesh=mesh,
        out_type=jax.ShapeDtypeStruct((total_rows, D), jnp.float32),
        scratch_types=[
            pltpu.VMEM((MAXLEN, D), jnp.float32),
            pltpu.VMEM((G, GB), jnp.int32),
            pltpu.VMEM((CHUNK, D), jnp.float32),
            pltpu.SemaphoreType.DMA,
        ],
        compiler_params=pltpu.CompilerParams(use_tc_tiling_on_sc=False),
    )
    def k(x_hbm, tok_hbm, pos_hbm, out_hbm, pos_v, idx_v, rows_v, sem):
        wid = lax.axis_index("s") * NC + lax.axis_index("c")
        pltpu.sync_copy(pos_hbm, pos_v)

        def chunk_body(c, carry):
            pltpu.sync_copy(x_hbm.at[wid, c], idx_v)
            handles = [
                pltpu.async_copy(tok_hbm.at[idx_v.at[j]],
                                 rows_v.at[pl.ds(j * GB, GB)], sem)
                for j in range(G)
            ]
            for h in handles:
                h.wait()

            def add_body(s, carry2):
                for cc in range(D // 16):
                    sl = pl.ds(cc * 16, 16)
                    pv = pos_v[s, sl]
                    rows_v[s, sl] += pv
                    rows_v[s + MAXLEN, sl] += pv
                return carry2

            lax.fori_loop(0, MAXLEN, add_body, 0)
            base = (wid * chunks + c) * CHUNK
            pltpu.sync_copy(rows_v, out_hbm.at[pl.ds(base, CHUNK)])
            return carry

        lax.fori_loop(0, chunks, chunk_body, 0)

    return k(x_r, token_table, pos_table)


def kernel(x, token_table, pos_table):
    B, S = x.shape
    total = B * S
    x_r = x.reshape(NW, total // (NW * CHUNK), G, GB)
    out = _run(x_r, token_table, pos_table, total)
    return out.reshape(B, S, D)


# speedup vs baseline: 1.0836x; 1.0836x over previous
import functools

import jax
import jax.numpy as jnp
from jax import lax
from jax.experimental import pallas as pl
from jax.experimental.pallas import tpu as pltpu
from jax.experimental.pallas import tpu_sc as plsc

D = 64
MAXLEN = 200
NC = 2
NS = 16
NW = NC * NS
G = 4
GB = 100
CHUNK = G * GB


@functools.partial(jax.jit, static_argnums=(3,))
def _run(x_r, token_table, pos_table, total_rows):
    chunks = total_rows // (NW * CHUNK)

    mesh = plsc.VectorSubcoreMesh(core_axis_name="c", subcore_axis_name="s")

    @functools.partial(
        pl.kernel,
        mesh=mesh,
        out_type=jax.ShapeDtypeStruct((total_rows, D), jnp.float32),
        scratch_types=[
            pltpu.VMEM((MAXLEN, D), jnp.float32),
            pltpu.VMEM((G, GB), jnp.int32),
            pltpu.VMEM((G, GB), jnp.int32),
            pltpu.VMEM((CHUNK, D), jnp.float32),
            pltpu.VMEM((CHUNK, D), jnp.float32),
            pltpu.SemaphoreType.DMA,
            pltpu.SemaphoreType.DMA,
            pltpu.SemaphoreType.DMA,
            pltpu.SemaphoreType.DMA,
        ],
        compiler_params=pltpu.CompilerParams(use_tc_tiling_on_sc=False),
    )
    def k(x_hbm, tok_hbm, pos_hbm, out_hbm,
          pos_v, idx0, idx1, rows0, rows1, gsem0, gsem1, wsem0, wsem1):
        wid = lax.axis_index("s") * NC + lax.axis_index("c")
        idx = (idx0, idx1)
        rows = (rows0, rows1)
        gsem = (gsem0, gsem1)
        wsem = (wsem0, wsem1)
        pltpu.sync_copy(pos_hbm, pos_v)

        def fire_gathers(p, c):
            pltpu.sync_copy(x_hbm.at[wid, c], idx[p])
            for j in range(G):
                pltpu.async_copy(tok_hbm.at[idx[p].at[j]],
                                 rows[p].at[pl.ds(j * GB, GB)], gsem[p])

        def wait_gathers(p):
            for j in range(G):
                pltpu.make_async_copy(tok_hbm.at[idx[p].at[j]],
                                      rows[p].at[pl.ds(j * GB, GB)],
                                      gsem[p]).wait()

        def out_slice(c):
            return out_hbm.at[pl.ds((wid * chunks + c) * CHUNK, CHUNK)]

        def fire_write(p, c):
            pltpu.async_copy(rows[p], out_slice(c), wsem[p])

        def wait_write(p, c):
            pltpu.make_async_copy(rows[p], out_slice(c), wsem[p]).wait()

        def add_pos(rv):
            @plsc.parallel_loop(0, MAXLEN, unroll=4)
            def add_body(s):
                for cc in range(D // 16):
                    sl = pl.ds(cc * 16, 16)
                    pv = pos_v[s, sl]
                    rv[s, sl] += pv
                    rv[s + MAXLEN, sl] += pv

        fire_gathers(0, 0)

        def body(i, carry):
            c0 = 2 * i

            @pl.when(i >= 1)
            def _():
                wait_write(1, c0 - 1)

            fire_gathers(1, c0 + 1)
            wait_gathers(0)
            add_pos(rows0)
            fire_write(0, c0)

            @pl.when(i < chunks // 2 - 1)
            def _():
                wait_write(0, c0)
                fire_gathers(0, c0 + 2)

            wait_gathers(1)
            add_pos(rows1)
            fire_write(1, c0 + 1)
            return carry

        lax.fori_loop(0, chunks // 2, body, 0)
        wait_write(0, chunks - 2)
        wait_write(1, chunks - 1)

    return k(x_r, token_table, pos_table)


def kernel(x, token_table, pos_table):
    B, S = x.shape
    total = B * S
    x_r = x.reshape(NW, total // (NW * CHUNK), G, GB)
    out = _run(x_r, token_table, pos_table, total)
    return out.reshape(B, S, D)
